# baseline (speedup 1.0000x reference)
import math

import jax
import jax.numpy as jnp
from jax import lax
from jax.experimental import pallas as pl
from jax.experimental.pallas import tpu as pltpu
from jax.experimental.pallas import tpu_sc as plsc

D_MODEL = 512
SCALE = math.sqrt(D_MODEL)
LANES = 16

NUM_CORES = 2
NUM_SUBCORES = 16
NW = NUM_CORES * NUM_SUBCORES

BATCH = 4096
SEQ = 50
B_PER_W = BATCH // NW
SEQ_PAD = 56
NBUF = 4


def _sc_body(table_hbm, idx_hbm, out_hbm, idx_v, *rest):
  cid = lax.axis_index("c")
  sid = lax.axis_index("s")
  wid = sid * NUM_CORES + cid

  bufs = rest[:NBUF]
  gsems = rest[NBUF:2 * NBUF]
  osems = rest[2 * NBUF:]

  pltpu.sync_copy(idx_hbm.at[wid], idx_v)

  def gcp(c, b):
    return pltpu.make_async_copy(table_hbm.at[idx_v.at[c]], bufs[b], gsems[b])

  def ocp(c, b):
    return pltpu.make_async_copy(
        bufs[b], out_hbm.at[wid * B_PER_W + c], osems[b])

  for b in range(NBUF):
    gcp(b, b).start()

  def do_round(p, start_next):
    for b in range(NBUF):
      g = p * NBUF + b
      gcp(g, b).wait()

      buf = bufs[b]

      @plsc.parallel_loop(0, SEQ_PAD)
      def _(r):
        for j in range(D_MODEL // LANES):
          sl = pl.ds(j * LANES, LANES)
          buf[r, sl] = buf[r, sl] * SCALE

      ocp(g, b).start()
      if start_next:
        ocp(g, b).wait()
        gcp(g + NBUF, b).start()

  nrounds = B_PER_W // NBUF

  def loop_body(p, carry):
    do_round(p, True)
    return carry

  lax.fori_loop(0, nrounds - 1, loop_body, jnp.int32(0))
  do_round(nrounds - 1, False)

  for b in range(NBUF):
    ocp(B_PER_W - NBUF + b, b).wait()


def _make_sc_call():
  mesh = plsc.VectorSubcoreMesh(core_axis_name="c", subcore_axis_name="s")
  return pl.kernel(
      _sc_body,
      out_type=jax.ShapeDtypeStruct((BATCH, SEQ_PAD, D_MODEL), jnp.float32),
      mesh=mesh,
      scratch_types=(
          [pltpu.VMEM((B_PER_W, SEQ_PAD), jnp.int32)]
          + [pltpu.VMEM((SEQ_PAD, D_MODEL), jnp.float32)] * NBUF
          + [pltpu.SemaphoreType.DMA] * (2 * NBUF)
      ),
      name="embedding_gather_scale_sc",
  )


def kernel(x, table):
  idx = x.reshape(NW, B_PER_W, SEQ).astype(jnp.int32)
  idx = jnp.pad(idx, ((0, 0), (0, 0), (0, SEQ_PAD - SEQ)))
  out = _make_sc_call()(table, idx)
  return out[:, :SEQ, :]


# speedup vs baseline: 3.2006x; 3.2006x over previous
import math

import jax
import jax.numpy as jnp
from jax import lax
from jax.experimental import pallas as pl
from jax.experimental.pallas import tpu as pltpu
from jax.experimental.pallas import tpu_sc as plsc

D_MODEL = 512
SCALE = math.sqrt(D_MODEL)
LANES = 16

NUM_CORES = 2
NUM_SUBCORES = 16
NW = NUM_CORES * NUM_SUBCORES

BATCH = 4096
SEQ = 50
B_PER_W = BATCH // NW
SEQ_PAD = 56
NBUF = 4


def _sc_body(table_hbm, idx_hbm, out_hbm, idx_v, *rest):
  cid = lax.axis_index("c")
  sid = lax.axis_index("s")
  wid = sid * NUM_CORES + cid

  bufs = rest[:NBUF]
  gsems = rest[NBUF:2 * NBUF]
  osems = rest[2 * NBUF:]

  pltpu.sync_copy(idx_hbm.at[wid], idx_v)

  def gcp(c, b):
    return pltpu.make_async_copy(table_hbm.at[idx_v.at[c]], bufs[b], gsems[b])

  def ocp(c, b):
    return pltpu.make_async_copy(
        bufs[b], out_hbm.at[wid * B_PER_W + c], osems[b])

  for b in range(NBUF):
    gcp(b, b).start()

  def do_round(p, start_next):
    for b in range(NBUF):
      g = p * NBUF + b
      gcp(g, b).wait()

      buf = bufs[b]

      @plsc.parallel_loop(0, SEQ_PAD)
      def _(r):
        for j in range(D_MODEL // LANES):
          sl = pl.ds(j * LANES, LANES)
          buf[r, sl] = buf[r, sl] * SCALE

      ocp(g, b).start()
      if start_next:
        ocp(g, b).wait()
        gcp(g + NBUF, b).start()

  nrounds = B_PER_W // NBUF

  def loop_body(p, carry):
    do_round(p, True)
    return carry

  lax.fori_loop(0, nrounds - 1, loop_body, jnp.int32(0))
  do_round(nrounds - 1, False)

  for b in range(NBUF):
    ocp(B_PER_W - NBUF + b, b).wait()


def _make_sc_call():
  mesh = plsc.VectorSubcoreMesh(core_axis_name="c", subcore_axis_name="s")
  return pl.kernel(
      _sc_body,
      out_type=jax.ShapeDtypeStruct((BATCH, SEQ_PAD, D_MODEL), jnp.float32),
      mesh=mesh,
      scratch_types=(
          [pltpu.VMEM((B_PER_W, SEQ_PAD), jnp.int32)]
          + [pltpu.VMEM((SEQ_PAD, D_MODEL), jnp.float32)] * NBUF
          + [pltpu.SemaphoreType.DMA] * (2 * NBUF)
      ),
      name="embedding_gather_scale_sc",
  )


def kernel(x, table):
  idx = x.reshape(NW, B_PER_W, SEQ).astype(jnp.int32)
  idx = jnp.pad(idx, ((0, 0), (0, 0), (0, SEQ_PAD - SEQ)), mode="edge")
  out = _make_sc_call()(table, idx)
  return out[:, :SEQ, :]


# speedup vs baseline: 6.7219x; 2.1002x over previous
import math

import jax
import jax.numpy as jnp
from jax import lax
from jax.experimental import pallas as pl
from jax.experimental.pallas import tpu as pltpu
from jax.experimental.pallas import tpu_sc as plsc

D_MODEL = 512
SCALE = math.sqrt(D_MODEL)
LANES = 16

NUM_CORES = 2
NUM_SUBCORES = 16
NW = NUM_CORES * NUM_SUBCORES

BATCH = 4096
SEQ = 50
B_PER_W = BATCH // NW
CHUNK = 64
SPLITS = B_PER_W // CHUNK
NCHUNK = SEQ * SPLITS
NBUF = 2


def _sc_body(table_hbm, idx_hbm, out_hbm, idx_v, *rest):
  cid = lax.axis_index("c")
  sid = lax.axis_index("s")
  wid = sid * NUM_CORES + cid

  bufs = rest[:NBUF]
  gsems = rest[NBUF:2 * NBUF]
  osems = rest[2 * NBUF:]

  pltpu.sync_copy(idx_hbm.at[wid], idx_v)

  def gcp(c, b):
    return pltpu.make_async_copy(table_hbm.at[idx_v.at[c]], bufs[b], gsems[b])

  def ocp(c, b):
    s = c // SPLITS
    h = c % SPLITS
    return pltpu.make_async_copy(
        bufs[b], out_hbm.at[s, pl.ds(wid * B_PER_W + h * CHUNK, CHUNK)],
        osems[b])

  for b in range(NBUF):
    gcp(b, b).start()

  def do_round(p, start_next):
    for b in range(NBUF):
      g = p * NBUF + b
      gcp(g, b).wait()

      buf = bufs[b]

      @plsc.parallel_loop(0, CHUNK)
      def _(r):
        for j in range(D_MODEL // LANES):
          sl = pl.ds(j * LANES, LANES)
          buf[r, sl] = buf[r, sl] * SCALE

      ocp(g, b).start()
      if start_next:
        ocp(g, b).wait()
        gcp(g + NBUF, b).start()

  nrounds = NCHUNK // NBUF

  def loop_body(p, carry):
    do_round(p, True)
    return carry

  lax.fori_loop(0, nrounds - 1, loop_body, jnp.int32(0))
  do_round(nrounds - 1, False)

  for b in range(NBUF):
    ocp(NCHUNK - NBUF + b, b).wait()


def _make_sc_call():
  mesh = plsc.VectorSubcoreMesh(core_axis_name="c", subcore_axis_name="s")
  return pl.kernel(
      _sc_body,
      out_type=jax.ShapeDtypeStruct((SEQ, BATCH, D_MODEL), jnp.float32),
      mesh=mesh,
      scratch_types=(
          [pltpu.VMEM((NCHUNK, CHUNK), jnp.int32)]
          + [pltpu.VMEM((CHUNK, D_MODEL), jnp.float32)] * NBUF
          + [pltpu.SemaphoreType.DMA] * (2 * NBUF)
      ),
      name="embedding_gather_scale_sc",
  )


def kernel(x, table):
  idx = (
      x.astype(jnp.int32)
      .T.reshape(SEQ, NW, SPLITS, CHUNK)
      .transpose(1, 0, 2, 3)
      .reshape(NW, NCHUNK, CHUNK)
  )
  out = _make_sc_call()(table, idx)
  return out.transpose(1, 0, 2)
